# baseline (speedup 1.0000x reference)
import functools

import jax
import jax.numpy as jnp
from jax import lax
from jax.experimental import pallas as pl
from jax.experimental.pallas import tpu as pltpu
from jax.experimental.pallas import tpu_sc as plsc

N = 4096
TL = 50
BL = 200
D = 64
V = 100000
C = 1000
NW = 32
CH = N // NW


def _zero_acc(acc):
    zero = jnp.zeros((16,), jnp.float32)

    def body(e, _):
        acc[e, pl.ds(0, 16)] = zero
        acc[e, pl.ds(16, 16)] = zero
        acc[e, pl.ds(32, 16)] = zero
        acc[e, pl.ds(48, 16)] = zero
        return 0

    lax.fori_loop(0, CH, body, 0)


def _sc_pool_body(w_hbm, title_hbm, body_hbm, tsum_hbm, bsum_hbm,
                  tidx_u, bidx_u, tidx_v, bidx_v, acc_t, acc_b, sem_t, sem_b):
    wid = lax.axis_index("s") * 2 + lax.axis_index("c")
    base = wid * CH

    pltpu.sync_copy(title_hbm.at[pl.ds(base * TL, CH * TL)], tidx_u)
    pltpu.sync_copy(body_hbm.at[pl.ds(base * BL, CH * BL)], bidx_u)
    _zero_acc(acc_t)
    _zero_acc(acc_b)

    lane = lax.iota(jnp.int32, 16)

    def tpass(k, _):
        for e0 in range(CH // 16):
            flat = (e0 * 16 + lane) * TL + k
            tidx_v[k, pl.ds(e0 * 16, 16)] = plsc.load_gather(tidx_u, [flat])
        pltpu.async_copy(w_hbm.at[tidx_v.at[k]], acc_t, sem_t, add=True)
        return 0

    def bpass(k, _):
        for e0 in range(CH // 16):
            flat = (e0 * 16 + lane) * BL + k
            bidx_v[k, pl.ds(e0 * 16, 16)] = plsc.load_gather(bidx_u, [flat])
        pltpu.async_copy(w_hbm.at[bidx_v.at[k]], acc_b, sem_b, add=True)
        return 0

    lax.fori_loop(0, TL, tpass, 0)
    lax.fori_loop(0, BL, bpass, 0)

    def tdrain(k, _):
        pltpu.make_async_copy(w_hbm.at[tidx_v.at[0]], acc_t, sem_t).wait()
        return 0

    def bdrain(k, _):
        pltpu.make_async_copy(w_hbm.at[bidx_v.at[0]], acc_b, sem_b).wait()
        return 0

    lax.fori_loop(0, TL, tdrain, 0)
    lax.fori_loop(0, BL, bdrain, 0)

    pltpu.sync_copy(acc_t, tsum_hbm.at[pl.ds(base, CH)])
    pltpu.sync_copy(acc_b, bsum_hbm.at[pl.ds(base, CH)])


_sc_pool = functools.partial(
    pl.kernel,
    out_type=(
        jax.ShapeDtypeStruct((N, D), jnp.float32),
        jax.ShapeDtypeStruct((N, D), jnp.float32),
    ),
    mesh=plsc.VectorSubcoreMesh(core_axis_name="c", subcore_axis_name="s"),
    scratch_types=[
        pltpu.VMEM((CH * TL,), jnp.int32),
        pltpu.VMEM((CH * BL,), jnp.int32),
        pltpu.VMEM((TL, CH), jnp.int32),
        pltpu.VMEM((BL, CH), jnp.int32),
        pltpu.VMEM((CH, D), jnp.float32),
        pltpu.VMEM((CH, D), jnp.float32),
        pltpu.SemaphoreType.DMA,
        pltpu.SemaphoreType.DMA,
    ],
    compiler_params=pltpu.CompilerParams(use_tc_tiling_on_sc=False,
                                         needs_layout_passes=False),
)(_sc_pool_body)


def _head_body(tidx_ref, bidx_ref, ts_ref, bs_ref, c_ref, o_ref):
    tcnt = jnp.sum((tidx_ref[...] > 0).astype(jnp.float32), axis=1, keepdims=True)
    bcnt = jnp.sum((bidx_ref[...] > 0).astype(jnp.float32), axis=1, keepdims=True)
    que = 0.3 * ts_ref[...] / tcnt + 0.7 * bs_ref[...] / bcnt
    sc = lax.dot_general(c_ref[...], que, (((1,), (1,)), ((), ())),
                         preferred_element_type=jnp.float32)
    m = jnp.max(sc, axis=0, keepdims=True)
    e = jnp.exp(sc - m)
    o_ref[...] = e / jnp.sum(e, axis=0, keepdims=True)


_R = 2048


def _head(tidx, bidx, tsum, bsum, c_table):
    return pl.pallas_call(
        _head_body,
        out_shape=jax.ShapeDtypeStruct((C, N), jnp.float32),
        grid=(N // _R,),
        in_specs=[
            pl.BlockSpec((_R, TL), lambda i: (i, 0)),
            pl.BlockSpec((_R, BL), lambda i: (i, 0)),
            pl.BlockSpec((_R, D), lambda i: (i, 0)),
            pl.BlockSpec((_R, D), lambda i: (i, 0)),
            pl.BlockSpec((C, D), lambda i: (0, 0)),
        ],
        out_specs=pl.BlockSpec((C, _R), lambda i: (0, i)),
    )(tidx, bidx, tsum, bsum, c_table)


def kernel(title_int, body_int, user_int, w_table, c_table):
    t = title_int.astype(jnp.int32)
    b = body_int.astype(jnp.int32)
    tsum, bsum = _sc_pool(w_table, t.reshape(-1), b.reshape(-1))
    return _head(t, b, tsum, bsum, c_table).T


# speedup vs baseline: 1.0030x; 1.0030x over previous
import functools

import jax
import jax.numpy as jnp
from jax import lax
from jax.experimental import pallas as pl
from jax.experimental.pallas import tpu as pltpu
from jax.experimental.pallas import tpu_sc as plsc

N = 4096
TL = 50
BL = 200
D = 64
V = 100000
C = 1000
NW = 32
CH = N // NW


def _zero_acc(acc):
    zero = jnp.zeros((16,), jnp.float32)

    def body(e, _):
        acc[e, pl.ds(0, 16)] = zero
        acc[e, pl.ds(16, 16)] = zero
        acc[e, pl.ds(32, 16)] = zero
        acc[e, pl.ds(48, 16)] = zero
        return 0

    lax.fori_loop(0, CH, body, 0)


def _sc_pool_body(w_hbm, title_hbm, body_hbm, tsum_hbm, bsum_hbm,
                  tidx_u, bidx_u, tidx_v, bidx_v, acc_t, acc_b, sem_t, sem_b):
    wid = lax.axis_index("s") * 2 + lax.axis_index("c")
    base = wid * CH

    pltpu.sync_copy(title_hbm.at[pl.ds(base * TL, CH * TL)], tidx_u)
    pltpu.sync_copy(body_hbm.at[pl.ds(base * BL, CH * BL)], bidx_u)
    _zero_acc(acc_t)
    _zero_acc(acc_b)

    lane = lax.iota(jnp.int32, 16)

    def tpass(k, _):
        for e0 in range(CH // 16):
            flat = (e0 * 16 + lane) * TL + k
            tidx_v[k, pl.ds(e0 * 16, 16)] = plsc.load_gather(tidx_u, [flat])
        pltpu.async_copy(w_hbm.at[tidx_v.at[k]], acc_t, sem_t, add=True)
        return 0

    def bpass(k, _):
        for e0 in range(CH // 16):
            flat = (e0 * 16 + lane) * BL + k
            bidx_v[k, pl.ds(e0 * 16, 16)] = plsc.load_gather(bidx_u, [flat])
        pltpu.async_copy(w_hbm.at[bidx_v.at[k]], acc_b, sem_b, add=True)
        return 0

    lax.fori_loop(0, TL, tpass, 0)
    lax.fori_loop(0, BL, bpass, 0)

    def tdrain(k, _):
        pltpu.make_async_copy(w_hbm.at[tidx_v.at[0]], acc_t, sem_t).wait()
        return 0

    def bdrain(k, _):
        pltpu.make_async_copy(w_hbm.at[bidx_v.at[0]], acc_b, sem_b).wait()
        return 0

    lax.fori_loop(0, TL, tdrain, 0)
    lax.fori_loop(0, BL, bdrain, 0)

    pltpu.sync_copy(acc_t, tsum_hbm.at[pl.ds(base, CH)])
    pltpu.sync_copy(acc_b, bsum_hbm.at[pl.ds(base, CH)])


_sc_pool = functools.partial(
    pl.kernel,
    out_type=(
        jax.ShapeDtypeStruct((N, D), jnp.float32),
        jax.ShapeDtypeStruct((N, D), jnp.float32),
    ),
    mesh=plsc.VectorSubcoreMesh(core_axis_name="c", subcore_axis_name="s"),
    scratch_types=[
        pltpu.VMEM((CH * TL,), jnp.int32),
        pltpu.VMEM((CH * BL,), jnp.int32),
        pltpu.VMEM((TL, CH), jnp.int32),
        pltpu.VMEM((BL, CH), jnp.int32),
        pltpu.VMEM((CH, D), jnp.float32),
        pltpu.VMEM((CH, D), jnp.float32),
        pltpu.SemaphoreType.DMA,
        pltpu.SemaphoreType.DMA,
    ],
    compiler_params=pltpu.CompilerParams(use_tc_tiling_on_sc=False,
                                         needs_layout_passes=False),
)(_sc_pool_body)


def _head_body(tidx_ref, bidx_ref, ts_ref, bs_ref, c_ref, o_ref):
    tcnt = jnp.sum((tidx_ref[...] > 0).astype(jnp.float32), axis=1, keepdims=True)
    bcnt = jnp.sum((bidx_ref[...] > 0).astype(jnp.float32), axis=1, keepdims=True)
    que = 0.3 * ts_ref[...] / tcnt + 0.7 * bs_ref[...] / bcnt
    sc = lax.dot_general(c_ref[...], que, (((1,), (1,)), ((), ())),
                         preferred_element_type=jnp.float32)
    m = jnp.max(sc, axis=0, keepdims=True)
    e = jnp.exp(sc - m)
    o_ref[...] = e / jnp.sum(e, axis=0, keepdims=True)


_R = 1024


def _head(tidx, bidx, tsum, bsum, c_table):
    return pl.pallas_call(
        _head_body,
        out_shape=jax.ShapeDtypeStruct((C, N), jnp.float32),
        grid=(N // _R,),
        in_specs=[
            pl.BlockSpec((_R, TL), lambda i: (i, 0)),
            pl.BlockSpec((_R, BL), lambda i: (i, 0)),
            pl.BlockSpec((_R, D), lambda i: (i, 0)),
            pl.BlockSpec((_R, D), lambda i: (i, 0)),
            pl.BlockSpec((C, D), lambda i: (0, 0)),
        ],
        out_specs=pl.BlockSpec((C, _R), lambda i: (0, i)),
    )(tidx, bidx, tsum, bsum, c_table)


def kernel(title_int, body_int, user_int, w_table, c_table):
    t = title_int.astype(jnp.int32)
    b = body_int.astype(jnp.int32)
    tsum, bsum = _sc_pool(w_table, t.reshape(-1), b.reshape(-1))
    return _head(t, b, tsum, bsum, c_table).T
